# baseline (speedup 1.0000x reference)
import functools

import numpy as np
import jax
import jax.numpy as jnp
from jax.experimental import pallas as pl
from jax.experimental.pallas import tpu as pltpu

_NUM_P = 1024
_LO_N = np.float32(-0.9999999403953552)
_SQRT2 = np.float32(1.4142135381698608)

_ERFINV_A = [3.43273939e-07, -3.5233877e-06, -4.39150654e-06, 0.00021858087,
             -0.00125372503, -0.00417768164, 0.246640727, 1.50140941]
_ERFINV_B = [0.000100950558, 0.00134934322, -0.00367342844, 0.00573950773,
             -0.0076224613, 0.00943887047, 1.00167406, 2.83297682]


def _np_threefry(k0, k1, x0, x1):
    def rotl(v, r):
        return ((v << np.uint32(r)) | (v >> np.uint32(32 - r))).astype(np.uint32)
    x0 = np.asarray(x0, np.uint32).copy()
    x1 = np.asarray(x1, np.uint32).copy()
    k0 = np.uint32(k0)
    k1 = np.uint32(k1)
    ks2 = np.uint32(k0 ^ k1 ^ np.uint32(0x1BD11BDA))
    ks = [k0, k1, ks2]
    rots = [13, 15, 26, 6, 17, 29, 16, 24]
    x0 = (x0 + k0).astype(np.uint32)
    x1 = (x1 + k1).astype(np.uint32)
    for g in range(5):
        for r in (rots[0:4] if g % 2 == 0 else rots[4:8]):
            x0 = (x0 + x1).astype(np.uint32)
            x1 = rotl(x1, r)
            x1 = (x1 ^ x0).astype(np.uint32)
        x0 = (x0 + ks[(g + 1) % 3]).astype(np.uint32)
        x1 = (x1 + ks[(g + 2) % 3] + np.uint32(g + 1)).astype(np.uint32)
    return x0, x1


def _np_fold_in(key, data):
    o0, o1 = _np_threefry(key[0], key[1], np.array([0], np.uint32),
                          np.array([data], np.uint32))
    return np.array([o0[0], o1[0]], np.uint32)


def _step_keys(T):
    base = np.array([0, 42], np.uint32)
    kn = np.stack([_np_fold_in(base, 2 * t) for t in range(T)])
    kr = np.stack([_np_fold_in(base, 2 * t + 1) for t in range(T)])
    return kn.astype(np.int64).astype(np.int32), kr.astype(np.int64).astype(np.int32)


def _rotl(x, r):
    return jax.lax.shift_left(x, np.int32(r)) | jax.lax.shift_right_logical(
        x, np.int32(32 - r))


def _hash(k0, k1, cnt):
    ks2 = k0 ^ k1 ^ np.int32(0x1BD11BDA)
    ks = (k0, k1, ks2)
    ra = (13, 15, 26, 6)
    rb = (17, 29, 16, 24)
    x1 = cnt + k1
    x0 = x1 + k0
    x1 = _rotl(x1, 13) ^ x0
    for r in (15, 26, 6):
        x0 = x0 + x1
        x1 = _rotl(x1, r)
        x1 = x1 ^ x0
    x0 = x0 + ks[1]
    x1 = x1 + (ks[2] + np.int32(1))
    for g in range(1, 5):
        for r in (ra if g % 2 == 0 else rb):
            x0 = x0 + x1
            x1 = _rotl(x1, r)
            x1 = x1 ^ x0
        x0 = x0 + ks[(g + 1) % 3]
        x1 = x1 + (ks[(g + 2) % 3] + np.int32(g + 1))
    return x0 ^ x1


def _bits_to_unit(bits):
    m = jax.lax.shift_right_logical(bits, np.int32(9)) | np.int32(0x3F800000)
    return jax.lax.bitcast_convert_type(m, jnp.float32) - np.float32(1.0)


def _erfinv(x):
    w = -jnp.log1p(-x * x)
    wa = w - np.float32(2.5)
    pa = jnp.full_like(x, np.float32(2.81022636e-08))
    for c in _ERFINV_A:
        pa = pa * wa + np.float32(c)
    wb = jnp.sqrt(w) - np.float32(3.0)
    pb = jnp.full_like(x, np.float32(-0.000200214257))
    for c in _ERFINV_B:
        pb = pb * wb + np.float32(c)
    return jnp.where(w < np.float32(5.0), pa, pb) * x


def _pf_kernel(kn_ref, kr_ref, z_ref, obs_ref, out_ref, parts, newp, wts,
               minis, ident, *, P, D, T, PP, KL):
    b = pl.program_id(0)
    NKC = P // KL
    NPT = P // PP

    parts[...] = jnp.broadcast_to(z_ref[0], (D, P))
    wts[...] = jnp.full((1, P), np.float32(1.0 / P), jnp.float32)
    ident[...] = (jax.lax.broadcasted_iota(jnp.int32, (P, P), 0)
                  == jax.lax.broadcasted_iota(jnp.int32, (P, P), 1)
                  ).astype(jnp.float32)

    iota_nd = jax.lax.broadcasted_iota(jnp.int32, (D, KL), 0)
    iota_np = jax.lax.broadcasted_iota(jnp.int32, (D, KL), 1)
    cnt_n0 = iota_np * np.int32(D) + iota_nd

    iota_pp = jax.lax.broadcasted_iota(jnp.int32, (PP, KL), 0)
    iota_kk = jax.lax.broadcasted_iota(jnp.int32, (PP, KL), 1)
    cnt_c0 = iota_pp * np.int32(P) + iota_kk

    iota_ohk = jax.lax.broadcasted_iota(jnp.int32, (P, P), 0)

    def step(t, carry):
        kn0 = kn_ref[t, 0]
        kn1 = kn_ref[t, 1]
        kr0 = kr_ref[t, 0]
        kr1 = kr_ref[t, 1]

        nbase = b * np.int32(P * D)
        for c in range(P // KL):
            cnt = (nbase + np.int32(c * KL * D)) + cnt_n0
            f = _bits_to_unit(_hash(kn0, kn1, cnt))
            u = f * np.float32(2.0) + _LO_N
            noise = _SQRT2 * _erfinv(u)
            sl = slice(c * KL, (c + 1) * KL)
            parts[:, sl] = parts[:, sl] + np.float32(0.1) * noise

        pr = parts[...]
        obs_t = obs_ref[0, t]
        d2 = jnp.sum((pr - obs_t) ** 2, axis=0, keepdims=True)
        lik = jnp.exp(np.float32(-0.5) * d2) + np.float32(1e-8)
        w = wts[...] * lik + np.float32(1e-10)
        w = w / jnp.sum(w)
        wts[...] = w
        rw = np.float32(1.0) / w

        cbase = b * np.int32(P * P)

        def ptile(pt, tc):
            p0 = pt * PP
            acc_b = jnp.full((PP, KL), np.int32(0x7FFFFFFF), jnp.int32)
            acc_k = jnp.zeros((PP, KL), jnp.int32)
            for kc in range(NKC):
                k0 = kc * KL
                cnt = (cbase + p0 * np.int32(P) + np.int32(k0)) + cnt_c0
                f = _bits_to_unit(_hash(kr0, kr1, cnt))
                rwc = jax.lax.slice(rw, (0, k0), (1, k0 + KL))
                val = jnp.log(f) * rwc
                vb = jax.lax.bitcast_convert_type(val, jnp.int32)
                better = vb < acc_b
                acc_b = jnp.minimum(acc_b, vb)
                acc_k = jnp.where(better, np.int32(k0) + iota_kk, acc_k)
            mv = jnp.min(acc_b, axis=1, keepdims=True)
            cand = jnp.where(acc_b == mv, acc_k, np.int32(2**30))
            mini = jnp.min(cand, axis=1, keepdims=True)
            minis[pl.ds(p0, PP), :] = mini.astype(jnp.float32)
            return tc

        jax.lax.fori_loop(0, NPT, ptile, 0, unroll=False)

        mrow = jax.lax.dot_general(
            minis[...], ident[...], (((0,), (0,)), ((), ())),
            precision=jax.lax.Precision.HIGHEST,
            preferred_element_type=jnp.float32)
        onehot = (iota_ohk == mrow.astype(jnp.int32)).astype(jnp.float32)
        newp[...] = jax.lax.dot_general(
            pr, onehot, (((1,), (0,)), ((), ())),
            precision=jax.lax.Precision.HIGHEST,
            preferred_element_type=jnp.float32)
        parts[...] = newp[...]
        return carry

    jax.lax.fori_loop(0, T, step, 0, unroll=False)
    out_ref[0] = jnp.sum(parts[...], axis=1, keepdims=True) * np.float32(1.0 / P)


def _build(B, D, T, P, interpret=False):
    PP = min(128, P)
    KL = min(128, P)
    grid_spec = pltpu.PrefetchScalarGridSpec(
        num_scalar_prefetch=2,
        grid=(B,),
        in_specs=[
            pl.BlockSpec((1, D, 1), lambda b, *_: (b, 0, 0)),
            pl.BlockSpec((1, T, D, 1), lambda b, *_: (b, 0, 0, 0)),
        ],
        out_specs=pl.BlockSpec((1, D, 1), lambda b, *_: (b, 0, 0)),
        scratch_shapes=[
            pltpu.VMEM((D, P), jnp.float32),
            pltpu.VMEM((D, P), jnp.float32),
            pltpu.VMEM((1, P), jnp.float32),
            pltpu.VMEM((P, 1), jnp.float32),
            pltpu.VMEM((P, P), jnp.float32),
        ],
    )
    return pl.pallas_call(
        functools.partial(_pf_kernel, P=P, D=D, T=T, PP=PP, KL=KL),
        grid_spec=grid_spec,
        out_shape=jax.ShapeDtypeStruct((B, D, 1), jnp.float32),
        interpret=interpret,
    )


def _run(z, observation, P, interpret=False):
    B, D = z.shape
    T = observation.shape[2]
    kn, kr = _step_keys(T)
    obs_t = jnp.transpose(observation, (0, 2, 1))[:, :, :, None]
    call = _build(B, D, T, P, interpret=interpret)
    out = call(jnp.asarray(kn), jnp.asarray(kr), z[:, :, None], obs_t)
    return out[:, :, 0]


def kernel(z, observation):
    return _run(z, observation, _NUM_P)


# speedup vs baseline: 1.1165x; 1.1165x over previous
import functools

import numpy as np
import jax
import jax.numpy as jnp
from jax.experimental import pallas as pl
from jax.experimental.pallas import tpu as pltpu

_NUM_P = 1024
_LO_N = np.float32(-0.9999999403953552)
_SQRT2 = np.float32(1.4142135381698608)

_ERFINV_A = [3.43273939e-07, -3.5233877e-06, -4.39150654e-06, 0.00021858087,
             -0.00125372503, -0.00417768164, 0.246640727, 1.50140941]
_ERFINV_B = [0.000100950558, 0.00134934322, -0.00367342844, 0.00573950773,
             -0.0076224613, 0.00943887047, 1.00167406, 2.83297682]


def _np_threefry(k0, k1, x0, x1):
    def rotl(v, r):
        return ((v << np.uint32(r)) | (v >> np.uint32(32 - r))).astype(np.uint32)
    x0 = np.asarray(x0, np.uint32).copy()
    x1 = np.asarray(x1, np.uint32).copy()
    k0 = np.uint32(k0)
    k1 = np.uint32(k1)
    ks2 = np.uint32(k0 ^ k1 ^ np.uint32(0x1BD11BDA))
    ks = [k0, k1, ks2]
    rots = [13, 15, 26, 6, 17, 29, 16, 24]
    x0 = (x0 + k0).astype(np.uint32)
    x1 = (x1 + k1).astype(np.uint32)
    for g in range(5):
        for r in (rots[0:4] if g % 2 == 0 else rots[4:8]):
            x0 = (x0 + x1).astype(np.uint32)
            x1 = rotl(x1, r)
            x1 = (x1 ^ x0).astype(np.uint32)
        x0 = (x0 + ks[(g + 1) % 3]).astype(np.uint32)
        x1 = (x1 + ks[(g + 2) % 3] + np.uint32(g + 1)).astype(np.uint32)
    return x0, x1


def _np_fold_in(key, data):
    o0, o1 = _np_threefry(key[0], key[1], np.array([0], np.uint32),
                          np.array([data], np.uint32))
    return np.array([o0[0], o1[0]], np.uint32)


def _step_keys(T):
    base = np.array([0, 42], np.uint32)
    kn = np.stack([_np_fold_in(base, 2 * t) for t in range(T)])
    kr = np.stack([_np_fold_in(base, 2 * t + 1) for t in range(T)])
    return kn.astype(np.int64).astype(np.int32), kr.astype(np.int64).astype(np.int32)


def _rotl(x, r):
    return jax.lax.shift_left(x, np.int32(r)) | jax.lax.shift_right_logical(
        x, np.int32(32 - r))


def _hash(k0, k1, cnt):
    ks2 = k0 ^ k1 ^ np.int32(0x1BD11BDA)
    ks = (k0, k1, ks2)
    ra = (13, 15, 26, 6)
    rb = (17, 29, 16, 24)
    x1 = cnt + k1
    x0 = x1 + k0
    x1 = _rotl(x1, 13) ^ x0
    for r in (15, 26, 6):
        x0 = x0 + x1
        x1 = _rotl(x1, r)
        x1 = x1 ^ x0
    x0 = x0 + ks[1]
    x1 = x1 + (ks[2] + np.int32(1))
    for g in range(1, 5):
        for r in (ra if g % 2 == 0 else rb):
            x0 = x0 + x1
            x1 = _rotl(x1, r)
            x1 = x1 ^ x0
        x0 = x0 + ks[(g + 1) % 3]
        x1 = x1 + (ks[(g + 2) % 3] + np.int32(g + 1))
    return x0 ^ x1


def _bits_to_unit(bits):
    m = jax.lax.shift_right_logical(bits, np.int32(9)) | np.int32(0x3F800000)
    return jax.lax.bitcast_convert_type(m, jnp.float32) - np.float32(1.0)


def _erfinv(x):
    w = -jnp.log1p(-x * x)
    wa = w - np.float32(2.5)
    pa = jnp.full_like(x, np.float32(2.81022636e-08))
    for c in _ERFINV_A:
        pa = pa * wa + np.float32(c)
    wb = jnp.sqrt(w) - np.float32(3.0)
    pb = jnp.full_like(x, np.float32(-0.000200214257))
    for c in _ERFINV_B:
        pb = pb * wb + np.float32(c)
    return jnp.where(w < np.float32(5.0), pa, pb) * x


def _pf_kernel(kn_ref, kr_ref, z_ref, obs_ref, out_ref, parts, nbuf, wts,
               minis, ident, *, P, D, T, PP, KL):
    b = pl.program_id(0)
    NKC = P // KL
    NPT = P // PP

    parts[...] = jnp.broadcast_to(z_ref[0], (D, P))
    wts[...] = jnp.full((1, P), np.float32(1.0 / P), jnp.float32)
    ident[...] = (jax.lax.broadcasted_iota(jnp.int32, (P, P), 0)
                  == jax.lax.broadcasted_iota(jnp.int32, (P, P), 1)
                  ).astype(jnp.bfloat16)

    iota_nd = jax.lax.broadcasted_iota(jnp.int32, (D, KL), 0)
    iota_np = jax.lax.broadcasted_iota(jnp.int32, (D, KL), 1)
    cnt_n0 = iota_np * np.int32(D) + iota_nd

    iota_pp = jax.lax.broadcasted_iota(jnp.int32, (PP, KL), 0)
    iota_kk = jax.lax.broadcasted_iota(jnp.int32, (PP, KL), 1)
    cnt_c0 = iota_pp * np.int32(P) + iota_kk
    iota_pp_col = jax.lax.broadcasted_iota(jnp.int32, (PP, 1), 0)

    iota_ohk = jax.lax.broadcasted_iota(jnp.int32, (P, P), 0)

    def _dot(a, b, dims):
        return jax.lax.dot_general(a, b, (dims, ((), ())),
                                   preferred_element_type=jnp.float32)

    def gather_prev():
        pr0 = parts[...]
        p1 = pr0.astype(jnp.bfloat16)
        r1 = pr0 - p1.astype(jnp.float32)
        p2 = r1.astype(jnp.bfloat16)
        p3 = (r1 - p2.astype(jnp.float32)).astype(jnp.bfloat16)
        mf = minis[...]
        m1 = mf.astype(jnp.bfloat16)
        m2 = (mf - m1.astype(jnp.float32)).astype(jnp.bfloat16)
        idb = ident[...]
        mrow = (_dot(m1, idb, ((0,), (0,)))
                + _dot(m2, idb, ((0,), (0,))))
        onehot = (iota_ohk == mrow.astype(jnp.int32)).astype(jnp.bfloat16)
        parts[...] = (_dot(p1, onehot, ((1,), (0,)))
                      + _dot(p2, onehot, ((1,), (0,)))
                      + _dot(p3, onehot, ((1,), (0,))))

    def step(t, carry):
        kn0 = kn_ref[t, 0]
        kn1 = kn_ref[t, 1]
        kr0 = kr_ref[t, 0]
        kr1 = kr_ref[t, 1]

        @pl.when(t > 0)
        def _():
            gather_prev()

        nbase = b * np.int32(P * D)
        for c in range(P // KL):
            cnt = (nbase + np.int32(c * KL * D)) + cnt_n0
            f = _bits_to_unit(_hash(kn0, kn1, cnt))
            u = f * np.float32(2.0) + _LO_N
            noise = _SQRT2 * _erfinv(u)
            sl = slice(c * KL, (c + 1) * KL)
            nbuf[:, sl] = np.float32(0.1) * noise
        parts[...] = parts[...] + nbuf[...]

        pr = parts[...]
        obs_t = obs_ref[0, t]
        d2 = jnp.sum((pr - obs_t) ** 2, axis=0, keepdims=True)
        lik = jnp.exp(np.float32(-0.5) * d2) + np.float32(1e-8)
        w = wts[...] * lik + np.float32(1e-10)
        w = w / jnp.sum(w)
        wts[...] = w
        rw = np.float32(1.0) / w

        cbase = b * np.int32(P * P)

        def ptile(pt, tc):
            p0 = pt * PP
            tbase = cbase + p0 * np.int32(P)
            acc_b = jnp.full((PP, KL), np.int32(0x7FFFFFFF), jnp.int32)
            acc_c = jnp.zeros((PP, KL), jnp.int32)
            for kc in range(NKC):
                k0 = kc * KL
                cnt = (tbase + np.int32(k0)) + cnt_c0
                f = _bits_to_unit(_hash(kr0, kr1, cnt))
                rwc = jax.lax.slice(rw, (0, k0), (1, k0 + KL))
                val = jnp.log(f) * rwc
                vb = jax.lax.bitcast_convert_type(val, jnp.int32)
                better = vb < acc_b
                acc_b = jnp.minimum(acc_b, vb)
                acc_c = jnp.where(better, cnt, acc_c)
            mv = jnp.min(acc_b, axis=1, keepdims=True)
            cand = jnp.where(acc_b == mv, acc_c, np.int32(0x7FFFFFFF))
            mcnt = jnp.min(cand, axis=1, keepdims=True)
            mini = (mcnt - tbase) - iota_pp_col * np.int32(P)
            minis[pl.ds(p0, PP), :] = mini.astype(jnp.float32)
            return tc

        jax.lax.fori_loop(0, NPT, ptile, 0, unroll=False)
        return carry

    jax.lax.fori_loop(0, T, step, 0, unroll=False)
    gather_prev()
    out_ref[0] = jnp.sum(parts[...], axis=1, keepdims=True) * np.float32(1.0 / P)


def _build(B, D, T, P, interpret=False):
    PP = min(128, P)
    KL = min(128, P)
    grid_spec = pltpu.PrefetchScalarGridSpec(
        num_scalar_prefetch=2,
        grid=(B,),
        in_specs=[
            pl.BlockSpec((1, D, 1), lambda b, *_: (b, 0, 0)),
            pl.BlockSpec((1, T, D, 1), lambda b, *_: (b, 0, 0, 0)),
        ],
        out_specs=pl.BlockSpec((1, D, 1), lambda b, *_: (b, 0, 0)),
        scratch_shapes=[
            pltpu.VMEM((D, P), jnp.float32),
            pltpu.VMEM((D, P), jnp.float32),
            pltpu.VMEM((1, P), jnp.float32),
            pltpu.VMEM((P, 1), jnp.float32),
            pltpu.VMEM((P, P), jnp.bfloat16),
        ],
    )
    return pl.pallas_call(
        functools.partial(_pf_kernel, P=P, D=D, T=T, PP=PP, KL=KL),
        grid_spec=grid_spec,
        out_shape=jax.ShapeDtypeStruct((B, D, 1), jnp.float32),
        interpret=interpret,
    )


def _run(z, observation, P, interpret=False):
    B, D = z.shape
    T = observation.shape[2]
    kn, kr = _step_keys(T)
    obs_t = jnp.transpose(observation, (0, 2, 1))[:, :, :, None]
    call = _build(B, D, T, P, interpret=interpret)
    out = call(jnp.asarray(kn), jnp.asarray(kr), z[:, :, None], obs_t)
    return out[:, :, 0]


def kernel(z, observation):
    return _run(z, observation, _NUM_P)


# speedup vs baseline: 1.1656x; 1.0440x over previous
import functools

import numpy as np
import jax
import jax.numpy as jnp
from jax.experimental import pallas as pl
from jax.experimental.pallas import tpu as pltpu

_NUM_P = 1024
_LO_N = np.float32(-0.9999999403953552)
_SQRT2 = np.float32(1.4142135381698608)

_ERFINV_A = [3.43273939e-07, -3.5233877e-06, -4.39150654e-06, 0.00021858087,
             -0.00125372503, -0.00417768164, 0.246640727, 1.50140941]
_ERFINV_B = [0.000100950558, 0.00134934322, -0.00367342844, 0.00573950773,
             -0.0076224613, 0.00943887047, 1.00167406, 2.83297682]


def _np_threefry(k0, k1, x0, x1):
    def rotl(v, r):
        return ((v << np.uint32(r)) | (v >> np.uint32(32 - r))).astype(np.uint32)
    x0 = np.asarray(x0, np.uint32).copy()
    x1 = np.asarray(x1, np.uint32).copy()
    k0 = np.uint32(k0)
    k1 = np.uint32(k1)
    ks2 = np.uint32(k0 ^ k1 ^ np.uint32(0x1BD11BDA))
    ks = [k0, k1, ks2]
    rots = [13, 15, 26, 6, 17, 29, 16, 24]
    x0 = (x0 + k0).astype(np.uint32)
    x1 = (x1 + k1).astype(np.uint32)
    for g in range(5):
        for r in (rots[0:4] if g % 2 == 0 else rots[4:8]):
            x0 = (x0 + x1).astype(np.uint32)
            x1 = rotl(x1, r)
            x1 = (x1 ^ x0).astype(np.uint32)
        x0 = (x0 + ks[(g + 1) % 3]).astype(np.uint32)
        x1 = (x1 + ks[(g + 2) % 3] + np.uint32(g + 1)).astype(np.uint32)
    return x0, x1


def _np_fold_in(key, data):
    o0, o1 = _np_threefry(key[0], key[1], np.array([0], np.uint32),
                          np.array([data], np.uint32))
    return np.array([o0[0], o1[0]], np.uint32)


def _step_keys(T):
    base = np.array([0, 42], np.uint32)
    kn = np.stack([_np_fold_in(base, 2 * t) for t in range(T)])
    kr = np.stack([_np_fold_in(base, 2 * t + 1) for t in range(T)])
    return kn.astype(np.int64).astype(np.int32), kr.astype(np.int64).astype(np.int32)


def _rotl(x, r):
    return jax.lax.shift_left(x, np.int32(r)) | jax.lax.shift_right_logical(
        x, np.int32(32 - r))


def _hash(k0, k1, cnt):
    ks2 = k0 ^ k1 ^ np.int32(0x1BD11BDA)
    ks = (k0, k1, ks2)
    ra = (13, 15, 26, 6)
    rb = (17, 29, 16, 24)
    x1 = cnt + k1
    x0 = x1 + k0
    x1 = _rotl(x1, 13) ^ x0
    for r in (15, 26, 6):
        x0 = x0 + x1
        x1 = _rotl(x1, r)
        x1 = x1 ^ x0
    x0 = x0 + ks[1]
    x1 = x1 + (ks[2] + np.int32(1))
    for g in range(1, 5):
        for r in (ra if g % 2 == 0 else rb):
            x0 = x0 + x1
            x1 = _rotl(x1, r)
            x1 = x1 ^ x0
        x0 = x0 + ks[(g + 1) % 3]
        x1 = x1 + (ks[(g + 2) % 3] + np.int32(g + 1))
    return x0 ^ x1


def _bits_to_unit(bits):
    m = jax.lax.shift_right_logical(bits, np.int32(9)) | np.int32(0x3F800000)
    return jax.lax.bitcast_convert_type(m, jnp.float32) - np.float32(1.0)


def _erfinv(x):
    w = -jnp.log1p(-x * x)
    wa = w - np.float32(2.5)
    pa = jnp.full_like(x, np.float32(2.81022636e-08))
    for c in _ERFINV_A:
        pa = pa * wa + np.float32(c)
    wb = jnp.sqrt(w) - np.float32(3.0)
    pb = jnp.full_like(x, np.float32(-0.000200214257))
    for c in _ERFINV_B:
        pb = pb * wb + np.float32(c)
    return jnp.where(w < np.float32(5.0), pa, pb) * x


def _pf_kernel(kn_ref, kr_ref, z_ref, obs_ref, out_ref, parts, nbuf, wts,
               minis, *, P, D, T, PP, KL):
    b = pl.program_id(0)
    NKC = P // KL
    NPT = P // PP

    parts[...] = jnp.broadcast_to(z_ref[0], (D, P))
    wts[...] = jnp.full((1, P), np.float32(1.0 / P), jnp.float32)
    iota_nd = jax.lax.broadcasted_iota(jnp.int32, (D, KL), 0)
    iota_np = jax.lax.broadcasted_iota(jnp.int32, (D, KL), 1)
    cnt_n0 = iota_np * np.int32(D) + iota_nd

    iota_pp = jax.lax.broadcasted_iota(jnp.int32, (PP, KL), 0)
    iota_kk = jax.lax.broadcasted_iota(jnp.int32, (PP, KL), 1)
    cnt_c0 = iota_pp * np.int32(P) + iota_kk
    iota_pp_col = jax.lax.broadcasted_iota(jnp.int32, (PP, 1), 0)

    iota_ohk = jax.lax.broadcasted_iota(jnp.int32, (P, P), 1)

    def gather_prev():
        pr0 = parts[...]
        p1 = pr0.astype(jnp.bfloat16)
        r1 = pr0 - p1.astype(jnp.float32)
        p2 = r1.astype(jnp.bfloat16)
        p3 = (r1 - p2.astype(jnp.float32)).astype(jnp.bfloat16)
        p123 = jnp.concatenate([p1, p2, p3], axis=0)
        onehot = (iota_ohk == minis[...]).astype(jnp.bfloat16)
        g = jax.lax.dot_general(
            p123, onehot, ((((1,), (1,))), ((), ())),
            preferred_element_type=jnp.float32)
        parts[...] = (g[0:D] + g[D:2 * D]) + g[2 * D:3 * D]

    def step(t, carry):
        kn0 = kn_ref[t, 0]
        kn1 = kn_ref[t, 1]
        kr0 = kr_ref[t, 0]
        kr1 = kr_ref[t, 1]

        @pl.when(t > 0)
        def _():
            gather_prev()

        nbase = b * np.int32(P * D)
        for c in range(P // KL):
            cnt = (nbase + np.int32(c * KL * D)) + cnt_n0
            f = _bits_to_unit(_hash(kn0, kn1, cnt))
            u = f * np.float32(2.0) + _LO_N
            noise = _SQRT2 * _erfinv(u)
            sl = slice(c * KL, (c + 1) * KL)
            nbuf[:, sl] = np.float32(0.1) * noise
        parts[...] = parts[...] + nbuf[...]

        pr = parts[...]
        obs_t = obs_ref[0, t]
        d2 = jnp.sum((pr - obs_t) ** 2, axis=0, keepdims=True)
        lik = jnp.exp(np.float32(-0.5) * d2) + np.float32(1e-8)
        w = wts[...] * lik + np.float32(1e-10)
        w = w / jnp.sum(w)
        wts[...] = w
        rw = np.float32(1.0) / w

        cbase = b * np.int32(P * P)

        def ptile(pt, tc):
            p0 = pt * PP
            tbase = cbase + p0 * np.int32(P)
            acc_b = jnp.full((PP, KL), np.int32(0x7FFFFFFF), jnp.int32)
            acc_c = jnp.zeros((PP, KL), jnp.int32)
            for kc in range(NKC):
                k0 = kc * KL
                cnt = (tbase + np.int32(k0)) + cnt_c0
                f = _bits_to_unit(_hash(kr0, kr1, cnt))
                rwc = jax.lax.slice(rw, (0, k0), (1, k0 + KL))
                val = jnp.log(f) * rwc
                vb = jax.lax.bitcast_convert_type(val, jnp.int32)
                better = vb < acc_b
                acc_b = jnp.minimum(acc_b, vb)
                acc_c = jnp.where(better, cnt, acc_c)
            mv = jnp.min(acc_b, axis=1, keepdims=True)
            cand = jnp.where(acc_b == mv, acc_c, np.int32(0x7FFFFFFF))
            mcnt = jnp.min(cand, axis=1, keepdims=True)
            mini = (mcnt - tbase) - iota_pp_col * np.int32(P)
            minis[pl.ds(p0, PP), :] = mini
            return tc

        jax.lax.fori_loop(0, NPT, ptile, 0, unroll=False)
        return carry

    jax.lax.fori_loop(0, T, step, 0, unroll=False)
    gather_prev()
    out_ref[0] = jnp.sum(parts[...], axis=1, keepdims=True) * np.float32(1.0 / P)


def _build(B, D, T, P, interpret=False):
    PP = min(128, P)
    KL = min(128, P)
    grid_spec = pltpu.PrefetchScalarGridSpec(
        num_scalar_prefetch=2,
        grid=(B,),
        in_specs=[
            pl.BlockSpec((1, D, 1), lambda b, *_: (b, 0, 0)),
            pl.BlockSpec((1, T, D, 1), lambda b, *_: (b, 0, 0, 0)),
        ],
        out_specs=pl.BlockSpec((1, D, 1), lambda b, *_: (b, 0, 0)),
        scratch_shapes=[
            pltpu.VMEM((D, P), jnp.float32),
            pltpu.VMEM((D, P), jnp.float32),
            pltpu.VMEM((1, P), jnp.float32),
            pltpu.VMEM((P, 1), jnp.int32),
        ],
    )
    return pl.pallas_call(
        functools.partial(_pf_kernel, P=P, D=D, T=T, PP=PP, KL=KL),
        grid_spec=grid_spec,
        out_shape=jax.ShapeDtypeStruct((B, D, 1), jnp.float32),
        interpret=interpret,
    )


def _run(z, observation, P, interpret=False):
    B, D = z.shape
    T = observation.shape[2]
    kn, kr = _step_keys(T)
    obs_t = jnp.transpose(observation, (0, 2, 1))[:, :, :, None]
    call = _build(B, D, T, P, interpret=interpret)
    out = call(jnp.asarray(kn), jnp.asarray(kr), z[:, :, None], obs_t)
    return out[:, :, 0]


def kernel(z, observation):
    return _run(z, observation, _NUM_P)


# speedup vs baseline: 1.1657x; 1.0001x over previous
import functools

import numpy as np
import jax
import jax.numpy as jnp
from jax.experimental import pallas as pl
from jax.experimental.pallas import tpu as pltpu

_NUM_P = 1024
_LO_N = np.float32(-0.9999999403953552)
_SQRT2 = np.float32(1.4142135381698608)

_ERFINV_A = [3.43273939e-07, -3.5233877e-06, -4.39150654e-06, 0.00021858087,
             -0.00125372503, -0.00417768164, 0.246640727, 1.50140941]
_ERFINV_B = [0.000100950558, 0.00134934322, -0.00367342844, 0.00573950773,
             -0.0076224613, 0.00943887047, 1.00167406, 2.83297682]


def _np_threefry(k0, k1, x0, x1):
    def rotl(v, r):
        return ((v << np.uint32(r)) | (v >> np.uint32(32 - r))).astype(np.uint32)
    x0 = np.asarray(x0, np.uint32).copy()
    x1 = np.asarray(x1, np.uint32).copy()
    k0 = np.uint32(k0)
    k1 = np.uint32(k1)
    ks2 = np.uint32(k0 ^ k1 ^ np.uint32(0x1BD11BDA))
    ks = [k0, k1, ks2]
    rots = [13, 15, 26, 6, 17, 29, 16, 24]
    x0 = (x0 + k0).astype(np.uint32)
    x1 = (x1 + k1).astype(np.uint32)
    for g in range(5):
        for r in (rots[0:4] if g % 2 == 0 else rots[4:8]):
            x0 = (x0 + x1).astype(np.uint32)
            x1 = rotl(x1, r)
            x1 = (x1 ^ x0).astype(np.uint32)
        x0 = (x0 + ks[(g + 1) % 3]).astype(np.uint32)
        x1 = (x1 + ks[(g + 2) % 3] + np.uint32(g + 1)).astype(np.uint32)
    return x0, x1


def _np_fold_in(key, data):
    o0, o1 = _np_threefry(key[0], key[1], np.array([0], np.uint32),
                          np.array([data], np.uint32))
    return np.array([o0[0], o1[0]], np.uint32)


def _step_keys(T):
    base = np.array([0, 42], np.uint32)
    kn = np.stack([_np_fold_in(base, 2 * t) for t in range(T)])
    kr = np.stack([_np_fold_in(base, 2 * t + 1) for t in range(T)])
    return kn.astype(np.int64).astype(np.int32), kr.astype(np.int64).astype(np.int32)


def _rotl(x, r):
    return jax.lax.shift_left(x, np.int32(r)) | jax.lax.shift_right_logical(
        x, np.int32(32 - r))


def _hash(k0, k1, cnt):
    ks2 = k0 ^ k1 ^ np.int32(0x1BD11BDA)
    ks = (k0, k1, ks2)
    ra = (13, 15, 26, 6)
    rb = (17, 29, 16, 24)
    x1 = cnt + k1
    x0 = x1 + k0
    x1 = _rotl(x1, 13) ^ x0
    for r in (15, 26, 6):
        x0 = x0 + x1
        x1 = _rotl(x1, r)
        x1 = x1 ^ x0
    x0 = x0 + ks[1]
    x1 = x1 + (ks[2] + np.int32(1))
    for g in range(1, 5):
        for r in (ra if g % 2 == 0 else rb):
            x0 = x0 + x1
            x1 = _rotl(x1, r)
            x1 = x1 ^ x0
        x0 = x0 + ks[(g + 1) % 3]
        x1 = x1 + (ks[(g + 2) % 3] + np.int32(g + 1))
    return x0 ^ x1


def _bits_to_unit(bits):
    m = jax.lax.shift_right_logical(bits, np.int32(9)) | np.int32(0x3F800000)
    return jax.lax.bitcast_convert_type(m, jnp.float32) - np.float32(1.0)


def _erfinv(x):
    w = -jnp.log1p(-x * x)
    wa = w - np.float32(2.5)
    pa = jnp.full_like(x, np.float32(2.81022636e-08))
    for c in _ERFINV_A:
        pa = pa * wa + np.float32(c)
    wb = jnp.sqrt(w) - np.float32(3.0)
    pb = jnp.full_like(x, np.float32(-0.000200214257))
    for c in _ERFINV_B:
        pb = pb * wb + np.float32(c)
    return jnp.where(w < np.float32(5.0), pa, pb) * x


def _pf_kernel(kn_ref, kr_ref, z_ref, obs_ref, out_ref, parts, nbuf, wts,
               minis, cnts, *, P, D, T, PP, KL):
    b = pl.program_id(0)
    NKC = P // KL
    NPT = P // PP

    parts[...] = jnp.broadcast_to(z_ref[0], (D, P))
    wts[...] = jnp.full((1, P), np.float32(1.0 / P), jnp.float32)
    iota_nd = jax.lax.broadcasted_iota(jnp.int32, (D, KL), 0)
    iota_np = jax.lax.broadcasted_iota(jnp.int32, (D, KL), 1)
    cnt_n0 = iota_np * np.int32(D) + iota_nd

    iota_pp = jax.lax.broadcasted_iota(jnp.int32, (PP, KL), 0)
    iota_kk = jax.lax.broadcasted_iota(jnp.int32, (PP, KL), 1)
    cnts[...] = iota_pp * np.int32(P) + iota_kk
    iota_pp_col = jax.lax.broadcasted_iota(jnp.int32, (PP, 1), 0)

    iota_ohk = jax.lax.broadcasted_iota(jnp.int32, (P, P), 1)

    def gather_prev():
        pr0 = parts[...]
        p1 = pr0.astype(jnp.bfloat16)
        r1 = pr0 - p1.astype(jnp.float32)
        p2 = r1.astype(jnp.bfloat16)
        p3 = (r1 - p2.astype(jnp.float32)).astype(jnp.bfloat16)
        p123 = jnp.concatenate([p1, p2, p3], axis=0)
        onehot = (iota_ohk == minis[...]).astype(jnp.bfloat16)
        g = jax.lax.dot_general(
            p123, onehot, ((((1,), (1,))), ((), ())),
            preferred_element_type=jnp.float32)
        parts[...] = (g[0:D] + g[D:2 * D]) + g[2 * D:3 * D]

    def step(t, carry):
        kn0 = kn_ref[t, 0]
        kn1 = kn_ref[t, 1]
        kr0 = kr_ref[t, 0]
        kr1 = kr_ref[t, 1]

        @pl.when(t > 0)
        def _():
            gather_prev()

        nbase = b * np.int32(P * D)
        for c in range(P // KL):
            cnt = (nbase + np.int32(c * KL * D)) + cnt_n0
            f = _bits_to_unit(_hash(kn0, kn1, cnt))
            u = f * np.float32(2.0) + _LO_N
            noise = _SQRT2 * _erfinv(u)
            sl = slice(c * KL, (c + 1) * KL)
            nbuf[:, sl] = np.float32(0.1) * noise
        parts[...] = parts[...] + nbuf[...]

        pr = parts[...]
        obs_t = obs_ref[0, t]
        d2 = jnp.sum((pr - obs_t) ** 2, axis=0, keepdims=True)
        lik = jnp.exp(np.float32(-0.5) * d2) + np.float32(1e-8)
        w = wts[...] * lik + np.float32(1e-10)
        w = w / jnp.sum(w)
        wts[...] = w
        rw = np.float32(1.0) / w

        cbase = b * np.int32(P * P)

        def ptile(pt, tc):
            p0 = pt * PP
            tbase = cbase + p0 * np.int32(P)
            acc_b = jnp.full((PP, KL), np.int32(0x7FFFFFFF), jnp.int32)
            acc_c = jnp.zeros((PP, KL), jnp.int32)
            for kc in range(NKC):
                k0 = kc * KL
                cnt = (tbase + np.int32(k0)) + cnts[...]
                f = _bits_to_unit(_hash(kr0, kr1, cnt))
                rwc = jax.lax.slice(rw, (0, k0), (1, k0 + KL))
                val = jnp.log(f) * rwc
                vb = jax.lax.bitcast_convert_type(val, jnp.int32)
                better = vb < acc_b
                acc_b = jnp.minimum(acc_b, vb)
                acc_c = jnp.where(better, cnt, acc_c)
            mv = jnp.min(acc_b, axis=1, keepdims=True)
            cand = jnp.where(acc_b == mv, acc_c, np.int32(0x7FFFFFFF))
            mcnt = jnp.min(cand, axis=1, keepdims=True)
            mini = (mcnt - tbase) - iota_pp_col * np.int32(P)
            minis[pl.ds(p0, PP), :] = mini
            return tc

        jax.lax.fori_loop(0, NPT, ptile, 0, unroll=False)
        return carry

    jax.lax.fori_loop(0, T, step, 0, unroll=False)
    gather_prev()
    out_ref[0] = jnp.sum(parts[...], axis=1, keepdims=True) * np.float32(1.0 / P)


def _build(B, D, T, P, interpret=False):
    PP = min(128, P)
    KL = min(128, P)
    grid_spec = pltpu.PrefetchScalarGridSpec(
        num_scalar_prefetch=2,
        grid=(B,),
        in_specs=[
            pl.BlockSpec((1, D, 1), lambda b, *_: (b, 0, 0)),
            pl.BlockSpec((1, T, D, 1), lambda b, *_: (b, 0, 0, 0)),
        ],
        out_specs=pl.BlockSpec((1, D, 1), lambda b, *_: (b, 0, 0)),
        scratch_shapes=[
            pltpu.VMEM((D, P), jnp.float32),
            pltpu.VMEM((D, P), jnp.float32),
            pltpu.VMEM((1, P), jnp.float32),
            pltpu.VMEM((P, 1), jnp.int32),
            pltpu.VMEM((PP, KL), jnp.int32),
        ],
    )
    return pl.pallas_call(
        functools.partial(_pf_kernel, P=P, D=D, T=T, PP=PP, KL=KL),
        grid_spec=grid_spec,
        out_shape=jax.ShapeDtypeStruct((B, D, 1), jnp.float32),
        interpret=interpret,
    )


def _run(z, observation, P, interpret=False):
    B, D = z.shape
    T = observation.shape[2]
    kn, kr = _step_keys(T)
    obs_t = jnp.transpose(observation, (0, 2, 1))[:, :, :, None]
    call = _build(B, D, T, P, interpret=interpret)
    out = call(jnp.asarray(kn), jnp.asarray(kr), z[:, :, None], obs_t)
    return out[:, :, 0]


def kernel(z, observation):
    return _run(z, observation, _NUM_P)


# speedup vs baseline: 1.1736x; 1.0068x over previous
import functools

import numpy as np
import jax
import jax.numpy as jnp
from jax.experimental import pallas as pl
from jax.experimental.pallas import tpu as pltpu

_NUM_P = 1024
_LO_N = np.float32(-0.9999999403953552)
_SQRT2 = np.float32(1.4142135381698608)

_ERFINV_A = [3.43273939e-07, -3.5233877e-06, -4.39150654e-06, 0.00021858087,
             -0.00125372503, -0.00417768164, 0.246640727, 1.50140941]
_ERFINV_B = [0.000100950558, 0.00134934322, -0.00367342844, 0.00573950773,
             -0.0076224613, 0.00943887047, 1.00167406, 2.83297682]


def _np_threefry(k0, k1, x0, x1):
    def rotl(v, r):
        return ((v << np.uint32(r)) | (v >> np.uint32(32 - r))).astype(np.uint32)
    x0 = np.asarray(x0, np.uint32).copy()
    x1 = np.asarray(x1, np.uint32).copy()
    k0 = np.uint32(k0)
    k1 = np.uint32(k1)
    ks2 = np.uint32(k0 ^ k1 ^ np.uint32(0x1BD11BDA))
    ks = [k0, k1, ks2]
    rots = [13, 15, 26, 6, 17, 29, 16, 24]
    x0 = (x0 + k0).astype(np.uint32)
    x1 = (x1 + k1).astype(np.uint32)
    for g in range(5):
        for r in (rots[0:4] if g % 2 == 0 else rots[4:8]):
            x0 = (x0 + x1).astype(np.uint32)
            x1 = rotl(x1, r)
            x1 = (x1 ^ x0).astype(np.uint32)
        x0 = (x0 + ks[(g + 1) % 3]).astype(np.uint32)
        x1 = (x1 + ks[(g + 2) % 3] + np.uint32(g + 1)).astype(np.uint32)
    return x0, x1


def _np_fold_in(key, data):
    o0, o1 = _np_threefry(key[0], key[1], np.array([0], np.uint32),
                          np.array([data], np.uint32))
    return np.array([o0[0], o1[0]], np.uint32)


def _step_keys(T):
    base = np.array([0, 42], np.uint32)
    kn = np.stack([_np_fold_in(base, 2 * t) for t in range(T)])
    kr = np.stack([_np_fold_in(base, 2 * t + 1) for t in range(T)])
    return kn.astype(np.int64).astype(np.int32), kr.astype(np.int64).astype(np.int32)


def _rotl(x, r):
    return jax.lax.shift_left(x, np.int32(r)) | jax.lax.shift_right_logical(
        x, np.int32(32 - r))


def _hash(k0, k1, x1):
    ks2 = k0 ^ k1 ^ np.int32(0x1BD11BDA)
    ks = (k0, k1, ks2)
    ra = (13, 15, 26, 6)
    rb = (17, 29, 16, 24)
    x0 = x1 + k0
    x1 = _rotl(x1, 13) ^ x0
    for r in (15, 26, 6):
        x0 = x0 + x1
        x1 = _rotl(x1, r)
        x1 = x1 ^ x0
    x0 = x0 + ks[1]
    x1 = x1 + (ks[2] + np.int32(1))
    for g in range(1, 5):
        for r in (ra if g % 2 == 0 else rb):
            x0 = x0 + x1
            x1 = _rotl(x1, r)
            x1 = x1 ^ x0
        x0 = x0 + ks[(g + 1) % 3]
        x1 = x1 + (ks[(g + 2) % 3] + np.int32(g + 1))
    return x0 ^ x1


def _bits_to_unit(bits):
    m = jax.lax.shift_right_logical(bits, np.int32(9)) | np.int32(0x3F800000)
    return jax.lax.bitcast_convert_type(m, jnp.float32) - np.float32(1.0)


def _erfinv(x):
    w = -jnp.log1p(-x * x)
    wa = w - np.float32(2.5)
    pa = jnp.full_like(x, np.float32(2.81022636e-08))
    for c in _ERFINV_A:
        pa = pa * wa + np.float32(c)
    wb = jnp.sqrt(w) - np.float32(3.0)
    pb = jnp.full_like(x, np.float32(-0.000200214257))
    for c in _ERFINV_B:
        pb = pb * wb + np.float32(c)
    return jnp.where(w < np.float32(5.0), pa, pb) * x


def _pf_kernel(kn_ref, kr_ref, z_ref, obs_ref, out_ref, parts, nbuf, wts,
               minis, cnts, *, P, D, T, PP, KL):
    b = pl.program_id(0)
    NKC = P // KL
    NPT = P // PP

    parts[...] = jnp.broadcast_to(z_ref[0], (D, P))
    wts[...] = jnp.full((1, P), np.float32(1.0 / P), jnp.float32)
    iota_nd = jax.lax.broadcasted_iota(jnp.int32, (D, KL), 0)
    iota_np = jax.lax.broadcasted_iota(jnp.int32, (D, KL), 1)
    cnt_n0 = iota_np * np.int32(D) + iota_nd

    iota_pp = jax.lax.broadcasted_iota(jnp.int32, (PP, KL), 0)
    iota_kk = jax.lax.broadcasted_iota(jnp.int32, (PP, KL), 1)
    cnts[...] = iota_pp * np.int32(P) + iota_kk
    iota_pp_col = jax.lax.broadcasted_iota(jnp.int32, (PP, 1), 0)

    iota_ohk = jax.lax.broadcasted_iota(jnp.int32, (P, P), 1)

    def gather_prev():
        pr0 = parts[...]
        p1 = pr0.astype(jnp.bfloat16)
        r1 = pr0 - p1.astype(jnp.float32)
        p2 = r1.astype(jnp.bfloat16)
        p3 = (r1 - p2.astype(jnp.float32)).astype(jnp.bfloat16)
        p123 = jnp.concatenate([p1, p2, p3], axis=0)
        onehot = (iota_ohk == minis[...]).astype(jnp.bfloat16)
        g = jax.lax.dot_general(
            p123, onehot, ((((1,), (1,))), ((), ())),
            preferred_element_type=jnp.float32)
        parts[...] = (g[0:D] + g[D:2 * D]) + g[2 * D:3 * D]

    def step(t, carry):
        kn0 = kn_ref[t, 0]
        kn1 = kn_ref[t, 1]
        kr0 = kr_ref[t, 0]
        kr1 = kr_ref[t, 1]

        @pl.when(t > 0)
        def _():
            gather_prev()

        nbase = b * np.int32(P * D)
        for c in range(P // KL):
            x1 = (nbase + np.int32(c * KL * D) + kn1) + cnt_n0
            f = _bits_to_unit(_hash(kn0, kn1, x1))
            u = f * np.float32(2.0) + _LO_N
            noise = _SQRT2 * _erfinv(u)
            sl = slice(c * KL, (c + 1) * KL)
            nbuf[:, sl] = np.float32(0.1) * noise
        parts[...] = parts[...] + nbuf[...]

        pr = parts[...]
        obs_t = obs_ref[0, t]
        d2 = jnp.sum((pr - obs_t) ** 2, axis=0, keepdims=True)
        lik = jnp.exp(np.float32(-0.5) * d2) + np.float32(1e-8)
        w = wts[...] * lik + np.float32(1e-10)
        w = w / jnp.sum(w)
        wts[...] = w
        rw = np.float32(1.0) / w

        cbase = b * np.int32(P * P)

        def ptile(pt, tc):
            p0 = pt * PP
            tbase = cbase + p0 * np.int32(P)
            acc_b = jnp.full((PP, KL), np.int32(0x7FFFFFFF), jnp.int32)
            acc_c = jnp.zeros((PP, KL), jnp.int32)
            for kc in range(NKC):
                k0 = kc * KL
                x1 = (tbase + np.int32(k0) + kr1) + cnts[...]
                f = _bits_to_unit(_hash(kr0, kr1, x1))
                rwc = jax.lax.slice(rw, (0, k0), (1, k0 + KL))
                val = jnp.log(f) * rwc
                vb = jax.lax.bitcast_convert_type(val, jnp.int32)
                better = vb < acc_b
                acc_b = jnp.minimum(acc_b, vb)
                acc_c = jnp.where(better, x1, acc_c)
            mv = jnp.min(acc_b, axis=1, keepdims=True)
            cand = jnp.where(acc_b == mv, acc_c, np.int32(0x7FFFFFFF))
            mcnt = jnp.min(cand, axis=1, keepdims=True)
            mini = ((mcnt - kr1) - tbase) - iota_pp_col * np.int32(P)
            minis[pl.ds(p0, PP), :] = mini
            return tc

        jax.lax.fori_loop(0, NPT, ptile, 0, unroll=False)
        return carry

    jax.lax.fori_loop(0, T, step, 0, unroll=False)
    gather_prev()
    out_ref[0] = jnp.sum(parts[...], axis=1, keepdims=True) * np.float32(1.0 / P)


def _build(B, D, T, P, interpret=False):
    PP = min(128, P)
    KL = min(128, P)
    grid_spec = pltpu.PrefetchScalarGridSpec(
        num_scalar_prefetch=2,
        grid=(B,),
        in_specs=[
            pl.BlockSpec((1, D, 1), lambda b, *_: (b, 0, 0)),
            pl.BlockSpec((1, T, D, 1), lambda b, *_: (b, 0, 0, 0)),
        ],
        out_specs=pl.BlockSpec((1, D, 1), lambda b, *_: (b, 0, 0)),
        scratch_shapes=[
            pltpu.VMEM((D, P), jnp.float32),
            pltpu.VMEM((D, P), jnp.float32),
            pltpu.VMEM((1, P), jnp.float32),
            pltpu.VMEM((P, 1), jnp.int32),
            pltpu.VMEM((PP, KL), jnp.int32),
        ],
    )
    return pl.pallas_call(
        functools.partial(_pf_kernel, P=P, D=D, T=T, PP=PP, KL=KL),
        grid_spec=grid_spec,
        out_shape=jax.ShapeDtypeStruct((B, D, 1), jnp.float32),
        interpret=interpret,
    )


def _run(z, observation, P, interpret=False):
    B, D = z.shape
    T = observation.shape[2]
    kn, kr = _step_keys(T)
    obs_t = jnp.transpose(observation, (0, 2, 1))[:, :, :, None]
    call = _build(B, D, T, P, interpret=interpret)
    out = call(jnp.asarray(kn), jnp.asarray(kr), z[:, :, None], obs_t)
    return out[:, :, 0]


def kernel(z, observation):
    return _run(z, observation, _NUM_P)


# speedup vs baseline: 1.1834x; 1.0083x over previous
import functools

import numpy as np
import jax
import jax.numpy as jnp
from jax.experimental import pallas as pl
from jax.experimental.pallas import tpu as pltpu

_NUM_P = 1024
_LO_N = np.float32(-0.9999999403953552)
_SQRT2 = np.float32(1.4142135381698608)

_ERFINV_A = [3.43273939e-07, -3.5233877e-06, -4.39150654e-06, 0.00021858087,
             -0.00125372503, -0.00417768164, 0.246640727, 1.50140941]
_ERFINV_B = [0.000100950558, 0.00134934322, -0.00367342844, 0.00573950773,
             -0.0076224613, 0.00943887047, 1.00167406, 2.83297682]


def _np_threefry(k0, k1, x0, x1):
    def rotl(v, r):
        return ((v << np.uint32(r)) | (v >> np.uint32(32 - r))).astype(np.uint32)
    x0 = np.asarray(x0, np.uint32).copy()
    x1 = np.asarray(x1, np.uint32).copy()
    k0 = np.uint32(k0)
    k1 = np.uint32(k1)
    ks2 = np.uint32(k0 ^ k1 ^ np.uint32(0x1BD11BDA))
    ks = [k0, k1, ks2]
    rots = [13, 15, 26, 6, 17, 29, 16, 24]
    x0 = (x0 + k0).astype(np.uint32)
    x1 = (x1 + k1).astype(np.uint32)
    for g in range(5):
        for r in (rots[0:4] if g % 2 == 0 else rots[4:8]):
            x0 = (x0 + x1).astype(np.uint32)
            x1 = rotl(x1, r)
            x1 = (x1 ^ x0).astype(np.uint32)
        x0 = (x0 + ks[(g + 1) % 3]).astype(np.uint32)
        x1 = (x1 + ks[(g + 2) % 3] + np.uint32(g + 1)).astype(np.uint32)
    return x0, x1


def _np_fold_in(key, data):
    o0, o1 = _np_threefry(key[0], key[1], np.array([0], np.uint32),
                          np.array([data], np.uint32))
    return np.array([o0[0], o1[0]], np.uint32)


def _step_keys(T):
    base = np.array([0, 42], np.uint32)
    kn = np.stack([_np_fold_in(base, 2 * t) for t in range(T)])
    kr = np.stack([_np_fold_in(base, 2 * t + 1) for t in range(T)])
    return kn.astype(np.int64).astype(np.int32), kr.astype(np.int64).astype(np.int32)


def _rotl(x, r):
    return jax.lax.shift_left(x, np.int32(r)) | jax.lax.shift_right_logical(
        x, np.int32(32 - r))


def _hash(k0, k1, x1):
    ks2 = k0 ^ k1 ^ np.int32(0x1BD11BDA)
    ks = (k0, k1, ks2)
    ra = (13, 15, 26, 6)
    rb = (17, 29, 16, 24)
    x0 = x1 + k0
    x1 = _rotl(x1, 13) ^ x0
    for r in (15, 26, 6):
        x0 = x0 + x1
        x1 = _rotl(x1, r)
        x1 = x1 ^ x0
    x0 = x0 + ks[1]
    x1 = x1 + (ks[2] + np.int32(1))
    for g in range(1, 5):
        for r in (ra if g % 2 == 0 else rb):
            x0 = x0 + x1
            x1 = _rotl(x1, r)
            x1 = x1 ^ x0
        x0 = x0 + ks[(g + 1) % 3]
        x1 = x1 + (ks[(g + 2) % 3] + np.int32(g + 1))
    return x0 ^ x1


def _bits_to_unit(bits):
    m = jax.lax.shift_right_logical(bits, np.int32(9)) | np.int32(0x3F800000)
    return jax.lax.bitcast_convert_type(m, jnp.float32) - np.float32(1.0)


def _erfinv(x):
    w = -jnp.log1p(-x * x)
    wa = w - np.float32(2.5)
    pa = jnp.full_like(x, np.float32(2.81022636e-08))
    for c in _ERFINV_A:
        pa = pa * wa + np.float32(c)
    wb = jnp.sqrt(w) - np.float32(3.0)
    pb = jnp.full_like(x, np.float32(-0.000200214257))
    for c in _ERFINV_B:
        pb = pb * wb + np.float32(c)
    return jnp.where(w < np.float32(5.0), pa, pb) * x


def _pf_kernel(kn_ref, kr_ref, zA, oA, zB, oB, outA, outB,
               pA, nA, wA, mA, pB, nB, wB, mB, cnts, *, P, D, T, PP, KL):
    pid = pl.program_id(0)
    NKC = P // KL
    NPT = P // PP
    S = ((zA, oA, outA, pA, nA, wA, mA, pid * np.int32(2)),
         (zB, oB, outB, pB, nB, wB, mB, pid * np.int32(2) + np.int32(1)))

    iota_nd = jax.lax.broadcasted_iota(jnp.int32, (D, KL), 0)
    iota_np = jax.lax.broadcasted_iota(jnp.int32, (D, KL), 1)
    cnt_n0 = iota_np * np.int32(D) + iota_nd

    iota_pp = jax.lax.broadcasted_iota(jnp.int32, (PP, KL), 0)
    iota_kk = jax.lax.broadcasted_iota(jnp.int32, (PP, KL), 1)
    cnts[...] = iota_pp * np.int32(P) + iota_kk
    iota_pp_col = jax.lax.broadcasted_iota(jnp.int32, (PP, 1), 0)

    iota_ohk = jax.lax.broadcasted_iota(jnp.int32, (P, P), 1)

    for (z_ref, obs_ref, out_ref, parts, nbuf, wts, minis, bidx) in S:
        parts[...] = jnp.broadcast_to(z_ref[0], (D, P))
        wts[...] = jnp.full((1, P), np.float32(1.0 / P), jnp.float32)

    def gather_prev(parts, minis):
        pr0 = parts[...]
        p1 = pr0.astype(jnp.bfloat16)
        r1 = pr0 - p1.astype(jnp.float32)
        p2 = r1.astype(jnp.bfloat16)
        p3 = (r1 - p2.astype(jnp.float32)).astype(jnp.bfloat16)
        p123 = jnp.concatenate([p1, p2, p3], axis=0)
        onehot = (iota_ohk == minis[...]).astype(jnp.bfloat16)
        g = jax.lax.dot_general(
            p123, onehot, ((((1,), (1,))), ((), ())),
            preferred_element_type=jnp.float32)
        parts[...] = (g[0:D] + g[D:2 * D]) + g[2 * D:3 * D]

    def step(t, carry):
        kn0 = kn_ref[t, 0]
        kn1 = kn_ref[t, 1]
        kr0 = kr_ref[t, 0]
        kr1 = kr_ref[t, 1]

        @pl.when(t > 0)
        def _():
            for (z_ref, obs_ref, out_ref, parts, nbuf, wts, minis, bidx) in S:
                gather_prev(parts, minis)

        rws = []
        for (z_ref, obs_ref, out_ref, parts, nbuf, wts, minis, bidx) in S:
            nbase = bidx * np.int32(P * D)
            for c in range(P // KL):
                x1 = (nbase + np.int32(c * KL * D) + kn1) + cnt_n0
                f = _bits_to_unit(_hash(kn0, kn1, x1))
                u = f * np.float32(2.0) + _LO_N
                noise = _SQRT2 * _erfinv(u)
                sl = slice(c * KL, (c + 1) * KL)
                nbuf[:, sl] = np.float32(0.1) * noise
            parts[...] = parts[...] + nbuf[...]

            pr = parts[...]
            obs_t = obs_ref[0, t]
            d2 = jnp.sum((pr - obs_t) ** 2, axis=0, keepdims=True)
            lik = jnp.exp(np.float32(-0.5) * d2) + np.float32(1e-8)
            w = wts[...] * lik + np.float32(1e-10)
            w = w / jnp.sum(w)
            wts[...] = w
            rws.append(np.float32(1.0) / w)

        for (z_ref, obs_ref, out_ref, parts, nbuf, wts, minis, bidx), rw in zip(S, rws):
            cbase = bidx * np.int32(P * P)

            def ptile(pt, tc, cbase=cbase, rw=rw, minis=minis, kr1=kr1):
                p0 = pt * PP
                tbase = cbase + p0 * np.int32(P)
                acc_b = jnp.full((PP, KL), np.int32(0x7FFFFFFF), jnp.int32)
                acc_c = jnp.zeros((PP, KL), jnp.int32)
                for kc in range(NKC):
                    k0 = kc * KL
                    x1 = (tbase + np.int32(k0) + kr1) + cnts[...]
                    f = _bits_to_unit(_hash(kr0, kr1, x1))
                    rwc = jax.lax.slice(rw, (0, k0), (1, k0 + KL))
                    val = jnp.log(f) * rwc
                    vb = jax.lax.bitcast_convert_type(val, jnp.int32)
                    better = vb < acc_b
                    acc_b = jnp.minimum(acc_b, vb)
                    acc_c = jnp.where(better, x1, acc_c)
                mv = jnp.min(acc_b, axis=1, keepdims=True)
                cand = jnp.where(acc_b == mv, acc_c, np.int32(0x7FFFFFFF))
                mcnt = jnp.min(cand, axis=1, keepdims=True)
                mini = ((mcnt - kr1) - tbase) - iota_pp_col * np.int32(P)
                minis[pl.ds(p0, PP), :] = mini
                return tc

            jax.lax.fori_loop(0, NPT, ptile, 0, unroll=False)
        return carry

    jax.lax.fori_loop(0, T, step, 0, unroll=False)
    for (z_ref, obs_ref, out_ref, parts, nbuf, wts, minis, bidx) in S:
        gather_prev(parts, minis)
        out_ref[0] = jnp.sum(parts[...], axis=1,
                             keepdims=True) * np.float32(1.0 / P)


def _build(B, D, T, P, interpret=False):
    PP = min(128, P)
    KL = min(128, P)
    BH = B // 2
    grid_spec = pltpu.PrefetchScalarGridSpec(
        num_scalar_prefetch=2,
        grid=(BH,),
        in_specs=[
            pl.BlockSpec((1, D, 1), lambda b, *_: (2 * b, 0, 0)),
            pl.BlockSpec((1, T, D, 1), lambda b, *_: (2 * b, 0, 0, 0)),
            pl.BlockSpec((1, D, 1), lambda b, *_: (2 * b + 1, 0, 0)),
            pl.BlockSpec((1, T, D, 1), lambda b, *_: (2 * b + 1, 0, 0, 0)),
        ],
        out_specs=[
            pl.BlockSpec((1, D, 1), lambda b, *_: (b, 0, 0)),
            pl.BlockSpec((1, D, 1), lambda b, *_: (b, 0, 0)),
        ],
        scratch_shapes=[
            pltpu.VMEM((D, P), jnp.float32),
            pltpu.VMEM((D, P), jnp.float32),
            pltpu.VMEM((1, P), jnp.float32),
            pltpu.VMEM((P, 1), jnp.int32),
            pltpu.VMEM((D, P), jnp.float32),
            pltpu.VMEM((D, P), jnp.float32),
            pltpu.VMEM((1, P), jnp.float32),
            pltpu.VMEM((P, 1), jnp.int32),
            pltpu.VMEM((PP, KL), jnp.int32),
        ],
    )
    return pl.pallas_call(
        functools.partial(_pf_kernel, P=P, D=D, T=T, PP=PP, KL=KL),
        grid_spec=grid_spec,
        out_shape=[jax.ShapeDtypeStruct((BH, D, 1), jnp.float32),
                   jax.ShapeDtypeStruct((BH, D, 1), jnp.float32)],
        interpret=interpret,
    )


def _run(z, observation, P, interpret=False):
    B, D = z.shape
    T = observation.shape[2]
    kn, kr = _step_keys(T)
    obs_t = jnp.transpose(observation, (0, 2, 1))[:, :, :, None]
    call = _build(B, D, T, P, interpret=interpret)
    z3 = z[:, :, None]
    outA, outB = call(jnp.asarray(kn), jnp.asarray(kr), z3, obs_t, z3, obs_t)
    out = jnp.stack([outA, outB], axis=1).reshape(B, D, 1)
    return out[:, :, 0]


def kernel(z, observation):
    return _run(z, observation, _NUM_P)


# speedup vs baseline: 1.2603x; 1.0650x over previous
import functools

import numpy as np
import jax
import jax.numpy as jnp
from jax.experimental import pallas as pl
from jax.experimental.pallas import tpu as pltpu

_NUM_P = 1024
_LO_N = np.float32(-0.9999999403953552)
_SQRT2 = np.float32(1.4142135381698608)

_ERFINV_A = [3.43273939e-07, -3.5233877e-06, -4.39150654e-06, 0.00021858087,
             -0.00125372503, -0.00417768164, 0.246640727, 1.50140941]
_ERFINV_B = [0.000100950558, 0.00134934322, -0.00367342844, 0.00573950773,
             -0.0076224613, 0.00943887047, 1.00167406, 2.83297682]


def _np_threefry(k0, k1, x0, x1):
    def rotl(v, r):
        return ((v << np.uint32(r)) | (v >> np.uint32(32 - r))).astype(np.uint32)
    x0 = np.asarray(x0, np.uint32).copy()
    x1 = np.asarray(x1, np.uint32).copy()
    k0 = np.uint32(k0)
    k1 = np.uint32(k1)
    ks2 = np.uint32(k0 ^ k1 ^ np.uint32(0x1BD11BDA))
    ks = [k0, k1, ks2]
    rots = [13, 15, 26, 6, 17, 29, 16, 24]
    x0 = (x0 + k0).astype(np.uint32)
    x1 = (x1 + k1).astype(np.uint32)
    for g in range(5):
        for r in (rots[0:4] if g % 2 == 0 else rots[4:8]):
            x0 = (x0 + x1).astype(np.uint32)
            x1 = rotl(x1, r)
            x1 = (x1 ^ x0).astype(np.uint32)
        x0 = (x0 + ks[(g + 1) % 3]).astype(np.uint32)
        x1 = (x1 + ks[(g + 2) % 3] + np.uint32(g + 1)).astype(np.uint32)
    return x0, x1


def _np_fold_in(key, data):
    o0, o1 = _np_threefry(key[0], key[1], np.array([0], np.uint32),
                          np.array([data], np.uint32))
    return np.array([o0[0], o1[0]], np.uint32)


def _step_keys(T):
    base = np.array([0, 42], np.uint32)
    kn = np.stack([_np_fold_in(base, 2 * t) for t in range(T)])
    kr = np.stack([_np_fold_in(base, 2 * t + 1) for t in range(T)])
    return kn.astype(np.int64).astype(np.int32), kr.astype(np.int64).astype(np.int32)


def _rotl(x, r):
    return jax.lax.shift_left(x, np.int32(r)) | jax.lax.shift_right_logical(
        x, np.int32(32 - r))


def _hash(k0, k1, x1):
    ks2 = k0 ^ k1 ^ np.int32(0x1BD11BDA)
    ks = (k0, k1, ks2)
    ra = (13, 15, 26, 6)
    rb = (17, 29, 16, 24)
    x0 = x1 + k0
    x1 = _rotl(x1, 13) ^ x0
    for r in (15, 26, 6):
        x0 = x0 + x1
        x1 = _rotl(x1, r)
        x1 = x1 ^ x0
    x0 = x0 + ks[1]
    x1 = x1 + (ks[2] + np.int32(1))
    for g in range(1, 5):
        for r in (ra if g % 2 == 0 else rb):
            x0 = x0 + x1
            x1 = _rotl(x1, r)
            x1 = x1 ^ x0
        x0 = x0 + ks[(g + 1) % 3]
        x1 = x1 + (ks[(g + 2) % 3] + np.int32(g + 1))
    return x0 ^ x1


def _bits_to_unit(bits):
    m = jax.lax.shift_right_logical(bits, np.int32(9)) | np.int32(0x3F800000)
    return jax.lax.bitcast_convert_type(m, jnp.float32) - np.float32(1.0)


def _erfinv(x):
    w = -jnp.log1p(-x * x)
    wa = w - np.float32(2.5)
    pa = jnp.full_like(x, np.float32(2.81022636e-08))
    for c in _ERFINV_A:
        pa = pa * wa + np.float32(c)
    wb = jnp.sqrt(w) - np.float32(3.0)
    pb = jnp.full_like(x, np.float32(-0.000200214257))
    for c in _ERFINV_B:
        pb = pb * wb + np.float32(c)
    return jnp.where(w < np.float32(5.0), pa, pb) * x


def _pf_kernel(kn_ref, kr_ref, zA, oA, zB, oB, outA, outB,
               pA, nA, wA, mA, pB, nB, wB, mB, cnts, *, P, D, T, PP, KL):
    pid = pl.program_id(0)
    NKC = P // KL
    NPT = P // PP
    S = ((zA, oA, outA, pA, nA, wA, mA, pid * np.int32(2)),
         (zB, oB, outB, pB, nB, wB, mB, pid * np.int32(2) + np.int32(1)))

    iota_nd = jax.lax.broadcasted_iota(jnp.int32, (D, KL), 0)
    iota_np = jax.lax.broadcasted_iota(jnp.int32, (D, KL), 1)
    cnt_n0 = iota_np * np.int32(D) + iota_nd

    iota_pp = jax.lax.broadcasted_iota(jnp.int32, (PP, KL), 0)
    iota_kk = jax.lax.broadcasted_iota(jnp.int32, (PP, KL), 1)
    cnts[...] = iota_pp * np.int32(P) + iota_kk
    iota_pp_col = jax.lax.broadcasted_iota(jnp.int32, (PP, 1), 0)

    iota_ohk = jax.lax.broadcasted_iota(jnp.int32, (P, P), 1)

    for (z_ref, obs_ref, out_ref, parts, nbuf, wts, minis, bidx) in S:
        parts[...] = jnp.broadcast_to(z_ref[0], (D, P))
        wts[...] = jnp.full((1, P), np.float32(1.0 / P), jnp.float32)

    def gather_prev(parts, minis):
        pr0 = parts[...]
        p1 = pr0.astype(jnp.bfloat16)
        r1 = pr0 - p1.astype(jnp.float32)
        p2 = r1.astype(jnp.bfloat16)
        p3 = (r1 - p2.astype(jnp.float32)).astype(jnp.bfloat16)
        p123 = jnp.concatenate([p1, p2, p3], axis=0)
        onehot = (iota_ohk == minis[...]).astype(jnp.bfloat16)
        g = jax.lax.dot_general(
            p123, onehot, ((((1,), (1,))), ((), ())),
            preferred_element_type=jnp.float32)
        parts[...] = (g[0:D] + g[D:2 * D]) + g[2 * D:3 * D]

    def step(t, carry):
        kn0 = kn_ref[t, 0]
        kn1 = kn_ref[t, 1]
        kr0 = kr_ref[t, 0]
        kr1 = kr_ref[t, 1]

        @pl.when(t > 0)
        def _():
            for (z_ref, obs_ref, out_ref, parts, nbuf, wts, minis, bidx) in S:
                gather_prev(parts, minis)

        rws = []
        for (z_ref, obs_ref, out_ref, parts, nbuf, wts, minis, bidx) in S:
            nbase = bidx * np.int32(P * D)
            for c in range(P // KL):
                x1 = (nbase + np.int32(c * KL * D) + kn1) + cnt_n0
                f = _bits_to_unit(_hash(kn0, kn1, x1))
                u = f * np.float32(2.0) + _LO_N
                noise = _SQRT2 * _erfinv(u)
                sl = slice(c * KL, (c + 1) * KL)
                nbuf[:, sl] = np.float32(0.1) * noise
            parts[...] = parts[...] + nbuf[...]

            pr = parts[...]
            obs_t = obs_ref[0, t]
            d2 = jnp.sum((pr - obs_t) ** 2, axis=0, keepdims=True)
            lik = jnp.exp(np.float32(-0.5) * d2) + np.float32(1e-8)
            w = wts[...] * lik + np.float32(1e-10)
            w = w / jnp.sum(w)
            wts[...] = w
            rws.append(np.float32(1.0) / w)

        for (z_ref, obs_ref, out_ref, parts, nbuf, wts, minis, bidx), rw in zip(S, rws):
            cbase = bidx * np.int32(P * P)

            def ptile(pt, tc, cbase=cbase, rw=rw, minis=minis, kr1=kr1):
                p0 = pt * PP
                tbase = cbase + p0 * np.int32(P)
                acc_b = jnp.full((PP, KL), np.int32(0x7FFFFFFF), jnp.int32)
                acc_c = jnp.zeros((PP, KL), jnp.int32)
                for kc in range(NKC):
                    k0 = kc * KL
                    x1 = (tbase + np.int32(k0) + kr1) + cnts[...]
                    f = _bits_to_unit(_hash(kr0, kr1, x1))
                    rwc = jax.lax.slice(rw, (0, k0), (1, k0 + KL))
                    val = jnp.log(f) * rwc
                    vb = jax.lax.bitcast_convert_type(val, jnp.int32)
                    better = vb < acc_b
                    acc_b = jnp.minimum(acc_b, vb)
                    acc_c = jnp.where(better, x1, acc_c)
                mv = jnp.min(acc_b, axis=1, keepdims=True)
                cand = jnp.where(acc_b == mv, acc_c, np.int32(0x7FFFFFFF))
                mcnt = jnp.min(cand, axis=1, keepdims=True)
                mini = ((mcnt - kr1) - tbase) - iota_pp_col * np.int32(P)
                minis[pl.ds(p0, PP), :] = mini
                return tc

            jax.lax.fori_loop(0, NPT, ptile, 0, unroll=2)
        return carry

    jax.lax.fori_loop(0, T, step, 0, unroll=False)
    for (z_ref, obs_ref, out_ref, parts, nbuf, wts, minis, bidx) in S:
        gather_prev(parts, minis)
        out_ref[0] = jnp.sum(parts[...], axis=1,
                             keepdims=True) * np.float32(1.0 / P)


def _build(B, D, T, P, interpret=False):
    PP = min(128, P)
    KL = min(128, P)
    BH = B // 2
    grid_spec = pltpu.PrefetchScalarGridSpec(
        num_scalar_prefetch=2,
        grid=(BH,),
        in_specs=[
            pl.BlockSpec((1, D, 1), lambda b, *_: (2 * b, 0, 0)),
            pl.BlockSpec((1, T, D, 1), lambda b, *_: (2 * b, 0, 0, 0)),
            pl.BlockSpec((1, D, 1), lambda b, *_: (2 * b + 1, 0, 0)),
            pl.BlockSpec((1, T, D, 1), lambda b, *_: (2 * b + 1, 0, 0, 0)),
        ],
        out_specs=[
            pl.BlockSpec((1, D, 1), lambda b, *_: (b, 0, 0)),
            pl.BlockSpec((1, D, 1), lambda b, *_: (b, 0, 0)),
        ],
        scratch_shapes=[
            pltpu.VMEM((D, P), jnp.float32),
            pltpu.VMEM((D, P), jnp.float32),
            pltpu.VMEM((1, P), jnp.float32),
            pltpu.VMEM((P, 1), jnp.int32),
            pltpu.VMEM((D, P), jnp.float32),
            pltpu.VMEM((D, P), jnp.float32),
            pltpu.VMEM((1, P), jnp.float32),
            pltpu.VMEM((P, 1), jnp.int32),
            pltpu.VMEM((PP, KL), jnp.int32),
        ],
    )
    return pl.pallas_call(
        functools.partial(_pf_kernel, P=P, D=D, T=T, PP=PP, KL=KL),
        grid_spec=grid_spec,
        out_shape=[jax.ShapeDtypeStruct((BH, D, 1), jnp.float32),
                   jax.ShapeDtypeStruct((BH, D, 1), jnp.float32)],
        interpret=interpret,
    )


def _run(z, observation, P, interpret=False):
    B, D = z.shape
    T = observation.shape[2]
    kn, kr = _step_keys(T)
    obs_t = jnp.transpose(observation, (0, 2, 1))[:, :, :, None]
    call = _build(B, D, T, P, interpret=interpret)
    z3 = z[:, :, None]
    outA, outB = call(jnp.asarray(kn), jnp.asarray(kr), z3, obs_t, z3, obs_t)
    out = jnp.stack([outA, outB], axis=1).reshape(B, D, 1)
    return out[:, :, 0]


def kernel(z, observation):
    return _run(z, observation, _NUM_P)


# speedup vs baseline: 1.2937x; 1.0265x over previous
import functools

import numpy as np
import jax
import jax.numpy as jnp
from jax.experimental import pallas as pl
from jax.experimental.pallas import tpu as pltpu

_NUM_P = 1024
_LO_N = np.float32(-0.9999999403953552)
_SQRT2 = np.float32(1.4142135381698608)

_ERFINV_A = [3.43273939e-07, -3.5233877e-06, -4.39150654e-06, 0.00021858087,
             -0.00125372503, -0.00417768164, 0.246640727, 1.50140941]
_ERFINV_B = [0.000100950558, 0.00134934322, -0.00367342844, 0.00573950773,
             -0.0076224613, 0.00943887047, 1.00167406, 2.83297682]


def _np_threefry(k0, k1, x0, x1):
    def rotl(v, r):
        return ((v << np.uint32(r)) | (v >> np.uint32(32 - r))).astype(np.uint32)
    x0 = np.asarray(x0, np.uint32).copy()
    x1 = np.asarray(x1, np.uint32).copy()
    k0 = np.uint32(k0)
    k1 = np.uint32(k1)
    ks2 = np.uint32(k0 ^ k1 ^ np.uint32(0x1BD11BDA))
    ks = [k0, k1, ks2]
    rots = [13, 15, 26, 6, 17, 29, 16, 24]
    x0 = (x0 + k0).astype(np.uint32)
    x1 = (x1 + k1).astype(np.uint32)
    for g in range(5):
        for r in (rots[0:4] if g % 2 == 0 else rots[4:8]):
            x0 = (x0 + x1).astype(np.uint32)
            x1 = rotl(x1, r)
            x1 = (x1 ^ x0).astype(np.uint32)
        x0 = (x0 + ks[(g + 1) % 3]).astype(np.uint32)
        x1 = (x1 + ks[(g + 2) % 3] + np.uint32(g + 1)).astype(np.uint32)
    return x0, x1


def _np_fold_in(key, data):
    o0, o1 = _np_threefry(key[0], key[1], np.array([0], np.uint32),
                          np.array([data], np.uint32))
    return np.array([o0[0], o1[0]], np.uint32)


def _step_keys(T):
    base = np.array([0, 42], np.uint32)
    kn = np.stack([_np_fold_in(base, 2 * t) for t in range(T)])
    kr = np.stack([_np_fold_in(base, 2 * t + 1) for t in range(T)])
    return kn.astype(np.int64).astype(np.int32), kr.astype(np.int64).astype(np.int32)


def _rotl(x, r):
    return jax.lax.shift_left(x, np.int32(r)) | jax.lax.shift_right_logical(
        x, np.int32(32 - r))


def _hash(k0, k1, x1):
    ks2 = k0 ^ k1 ^ np.int32(0x1BD11BDA)
    ks = (k0, k1, ks2)
    ra = (13, 15, 26, 6)
    rb = (17, 29, 16, 24)
    x0 = x1 + k0
    x1 = _rotl(x1, 13) ^ x0
    for r in (15, 26, 6):
        x0 = x0 + x1
        x1 = _rotl(x1, r)
        x1 = x1 ^ x0
    x0 = x0 + ks[1]
    x1 = x1 + (ks[2] + np.int32(1))
    for g in range(1, 5):
        for r in (ra if g % 2 == 0 else rb):
            x0 = x0 + x1
            x1 = _rotl(x1, r)
            x1 = x1 ^ x0
        x0 = x0 + ks[(g + 1) % 3]
        x1 = x1 + (ks[(g + 2) % 3] + np.int32(g + 1))
    return x0 ^ x1


def _bits_to_unit(bits):
    m = jax.lax.shift_right_logical(bits, np.int32(9)) | np.int32(0x3F800000)
    return jax.lax.bitcast_convert_type(m, jnp.float32) - np.float32(1.0)


def _erfinv(x):
    w = -jnp.log1p(-x * x)
    wa = w - np.float32(2.5)
    pa = jnp.full_like(x, np.float32(2.81022636e-08))
    for c in _ERFINV_A:
        pa = pa * wa + np.float32(c)
    wb = jnp.sqrt(w) - np.float32(3.0)
    pb = jnp.full_like(x, np.float32(-0.000200214257))
    for c in _ERFINV_B:
        pb = pb * wb + np.float32(c)
    return jnp.where(w < np.float32(5.0), pa, pb) * x


def _pf_kernel(kn_ref, kr_ref, zA, oA, zB, oB, outA, outB,
               pA, nA, wA, mA, pB, nB, wB, mB, cnts, *, P, D, T, PP, KL):
    pid = pl.program_id(0)
    NKC = P // KL
    NPT = P // PP
    S = ((zA, oA, outA, pA, nA, wA, mA, pid * np.int32(2)),
         (zB, oB, outB, pB, nB, wB, mB, pid * np.int32(2) + np.int32(1)))

    iota_nd = jax.lax.broadcasted_iota(jnp.int32, (D, KL), 0)
    iota_np = jax.lax.broadcasted_iota(jnp.int32, (D, KL), 1)
    cnt_n0 = iota_np * np.int32(D) + iota_nd

    iota_pp = jax.lax.broadcasted_iota(jnp.int32, (PP, KL), 0)
    iota_kk = jax.lax.broadcasted_iota(jnp.int32, (PP, KL), 1)
    cnts[...] = iota_pp * np.int32(P) + iota_kk
    iota_pp_col = jax.lax.broadcasted_iota(jnp.int32, (PP, 1), 0)

    iota_ohk = jax.lax.broadcasted_iota(jnp.int32, (P, P), 1)

    for (z_ref, obs_ref, out_ref, parts, nbuf, wts, minis, bidx) in S:
        parts[...] = jnp.broadcast_to(z_ref[0], (D, P))
        wts[...] = jnp.full((1, P), np.float32(1.0 / P), jnp.float32)

    def gather_prev(parts, minis):
        pr0 = parts[...]
        p1 = pr0.astype(jnp.bfloat16)
        r1 = pr0 - p1.astype(jnp.float32)
        p2 = r1.astype(jnp.bfloat16)
        p3 = (r1 - p2.astype(jnp.float32)).astype(jnp.bfloat16)
        p123 = jnp.concatenate([p1, p2, p3], axis=0)
        onehot = (iota_ohk == minis[...]).astype(jnp.bfloat16)
        g = jax.lax.dot_general(
            p123, onehot, ((((1,), (1,))), ((), ())),
            preferred_element_type=jnp.float32)
        parts[...] = (g[0:D] + g[D:2 * D]) + g[2 * D:3 * D]

    def step(t, carry):
        kn0 = kn_ref[t, 0]
        kn1 = kn_ref[t, 1]
        kr0 = kr_ref[t, 0]
        kr1 = kr_ref[t, 1]

        @pl.when(t > 0)
        def _():
            for (z_ref, obs_ref, out_ref, parts, nbuf, wts, minis, bidx) in S:
                gather_prev(parts, minis)

        rws = []
        for (z_ref, obs_ref, out_ref, parts, nbuf, wts, minis, bidx) in S:
            nbase = bidx * np.int32(P * D)
            for c in range(P // KL):
                x1 = (nbase + np.int32(c * KL * D) + kn1) + cnt_n0
                f = _bits_to_unit(_hash(kn0, kn1, x1))
                u = f * np.float32(2.0) + _LO_N
                noise = _SQRT2 * _erfinv(u)
                sl = slice(c * KL, (c + 1) * KL)
                nbuf[:, sl] = np.float32(0.1) * noise
            parts[...] = parts[...] + nbuf[...]

            pr = parts[...]
            obs_t = obs_ref[0, t]
            d2 = jnp.sum((pr - obs_t) ** 2, axis=0, keepdims=True)
            lik = jnp.exp(np.float32(-0.5) * d2) + np.float32(1e-8)
            w = wts[...] * lik + np.float32(1e-10)
            w = w / jnp.sum(w)
            wts[...] = w
            rws.append(np.float32(1.0) / w)

        for (z_ref, obs_ref, out_ref, parts, nbuf, wts, minis, bidx), rw in zip(S, rws):
            cbase = bidx * np.int32(P * P)

            def ptile(pt, tc, cbase=cbase, rw=rw, minis=minis, kr1=kr1):
                p0 = pt * PP
                tbase = cbase + p0 * np.int32(P)
                acc_b = jnp.full((PP, KL), np.int32(0x7FFFFFFF), jnp.int32)
                acc_c = jnp.zeros((PP, KL), jnp.int32)
                for kc in range(NKC):
                    k0 = kc * KL
                    x1 = (tbase + np.int32(k0) + kr1) + cnts[...]
                    f = _bits_to_unit(_hash(kr0, kr1, x1))
                    rwc = jax.lax.slice(rw, (0, k0), (1, k0 + KL))
                    val = jnp.log(f) * rwc
                    vb = jax.lax.bitcast_convert_type(val, jnp.int32)
                    better = vb < acc_b
                    acc_b = jnp.minimum(acc_b, vb)
                    acc_c = jnp.where(better, x1, acc_c)
                mv = jnp.min(acc_b, axis=1, keepdims=True)
                cand = jnp.where(acc_b == mv, acc_c, np.int32(0x7FFFFFFF))
                mcnt = jnp.min(cand, axis=1, keepdims=True)
                mini = ((mcnt - kr1) - tbase) - iota_pp_col * np.int32(P)
                minis[pl.ds(p0, PP), :] = mini
                return tc

            jax.lax.fori_loop(0, NPT, ptile, 0, unroll=4)
        return carry

    jax.lax.fori_loop(0, T, step, 0, unroll=False)
    for (z_ref, obs_ref, out_ref, parts, nbuf, wts, minis, bidx) in S:
        gather_prev(parts, minis)
        out_ref[0] = jnp.sum(parts[...], axis=1,
                             keepdims=True) * np.float32(1.0 / P)


def _build(B, D, T, P, interpret=False):
    PP = min(128, P)
    KL = min(128, P)
    BH = B // 2
    grid_spec = pltpu.PrefetchScalarGridSpec(
        num_scalar_prefetch=2,
        grid=(BH,),
        in_specs=[
            pl.BlockSpec((1, D, 1), lambda b, *_: (2 * b, 0, 0)),
            pl.BlockSpec((1, T, D, 1), lambda b, *_: (2 * b, 0, 0, 0)),
            pl.BlockSpec((1, D, 1), lambda b, *_: (2 * b + 1, 0, 0)),
            pl.BlockSpec((1, T, D, 1), lambda b, *_: (2 * b + 1, 0, 0, 0)),
        ],
        out_specs=[
            pl.BlockSpec((1, D, 1), lambda b, *_: (b, 0, 0)),
            pl.BlockSpec((1, D, 1), lambda b, *_: (b, 0, 0)),
        ],
        scratch_shapes=[
            pltpu.VMEM((D, P), jnp.float32),
            pltpu.VMEM((D, P), jnp.float32),
            pltpu.VMEM((1, P), jnp.float32),
            pltpu.VMEM((P, 1), jnp.int32),
            pltpu.VMEM((D, P), jnp.float32),
            pltpu.VMEM((D, P), jnp.float32),
            pltpu.VMEM((1, P), jnp.float32),
            pltpu.VMEM((P, 1), jnp.int32),
            pltpu.VMEM((PP, KL), jnp.int32),
        ],
    )
    return pl.pallas_call(
        functools.partial(_pf_kernel, P=P, D=D, T=T, PP=PP, KL=KL),
        grid_spec=grid_spec,
        out_shape=[jax.ShapeDtypeStruct((BH, D, 1), jnp.float32),
                   jax.ShapeDtypeStruct((BH, D, 1), jnp.float32)],
        interpret=interpret,
    )


def _run(z, observation, P, interpret=False):
    B, D = z.shape
    T = observation.shape[2]
    kn, kr = _step_keys(T)
    obs_t = jnp.transpose(observation, (0, 2, 1))[:, :, :, None]
    call = _build(B, D, T, P, interpret=interpret)
    z3 = z[:, :, None]
    outA, outB = call(jnp.asarray(kn), jnp.asarray(kr), z3, obs_t, z3, obs_t)
    out = jnp.stack([outA, outB], axis=1).reshape(B, D, 1)
    return out[:, :, 0]


def kernel(z, observation):
    return _run(z, observation, _NUM_P)
